# baseline (speedup 1.0000x reference)
import functools

import jax
import jax.numpy as jnp
from jax import lax
from jax.experimental import pallas as pl
from jax.experimental.pallas import tpu as pltpu
from jax.experimental.pallas import tpu_sc as plsc

_CH = 128


def kernel(pairs, init_emb, output_vecs):
    B = pairs.shape[0]
    D = init_emb.shape[1]
    info = plsc.get_sparse_core_info()
    nc, ns = info.num_cores, info.num_subcores
    nw = nc * ns
    b_per_w = B // nw
    nch = b_per_w // _CH

    src_idx = pairs[:, 0].astype(jnp.int32).reshape(nw, nch, _CH)
    dst_idx = pairs[:, 1].astype(jnp.int32).reshape(nw, nch, _CH)

    mesh = plsc.VectorSubcoreMesh(core_axis_name="c", subcore_axis_name="s")

    @functools.partial(
        pl.kernel,
        mesh=mesh,
        out_type=jax.ShapeDtypeStruct((B,), jnp.float32),
        compiler_params=pltpu.CompilerParams(
            needs_layout_passes=False, use_tc_tiling_on_sc=False),
        scratch_types=[
            pltpu.VMEM((nch, _CH), jnp.int32),
            pltpu.VMEM((nch, _CH), jnp.int32),
            pltpu.VMEM((b_per_w, D), jnp.float32),
            pltpu.VMEM((b_per_w, D), jnp.float32),
            pltpu.VMEM((b_per_w,), jnp.float32),
            pltpu.VMEM((16 * 16,), jnp.float32),
            pltpu.SemaphoreType.DMA,
        ],
    )
    def run(src_idx_hbm, dst_idx_hbm, src_tab_hbm, dst_tab_hbm, out_hbm,
            sidx_v, didx_v, srows_v, drows_v, out_v, prod_v, sem):
        wid = lax.axis_index("s") * nc + lax.axis_index("c")

        pltpu.sync_copy(src_idx_hbm.at[wid], sidx_v)
        pltpu.sync_copy(dst_idx_hbm.at[wid], didx_v)

        copies = []
        for k in range(nch):
            copies.append(pltpu.async_copy(
                src_tab_hbm.at[sidx_v.at[k]],
                srows_v.at[pl.ds(k * _CH, _CH)], sem))
            copies.append(pltpu.async_copy(
                dst_tab_hbm.at[didx_v.at[k]],
                drows_v.at[pl.ds(k * _CH, _CH)], sem))
        for c in copies:
            c.wait()

        iota16 = lax.iota(jnp.int32, 16)

        def body(g, _):
            rows = iota16 + g * 16
            acc = jnp.zeros((16,), jnp.float32)
            for c in range(D):
                col = jnp.full((16,), c, dtype=jnp.int32)
                a = plsc.load_gather(srows_v, [rows, col])
                b = plsc.load_gather(drows_v, [rows, col])
                acc = acc + a * b
            prob = 1.0 / (1.0 + jnp.exp(-acc))
            out_v[pl.ds(g * 16, 16)] = prob
            return 0

        lax.fori_loop(0, b_per_w // 16, body, 0)

        pltpu.sync_copy(out_v, out_hbm.at[pl.ds(wid * b_per_w, b_per_w)])

    return run(src_idx, dst_idx, init_emb, output_vecs)


# speedup vs baseline: 6.3216x; 6.3216x over previous
import functools

import jax
import jax.numpy as jnp
from jax import lax
from jax.experimental import pallas as pl
from jax.experimental.pallas import tpu as pltpu
from jax.experimental.pallas import tpu_sc as plsc

_RND = 16


def kernel(pairs, init_emb, output_vecs):
    B = pairs.shape[0]
    V, D = init_emb.shape
    info = plsc.get_sparse_core_info()
    nc, ns = info.num_cores, info.num_subcores
    nw = nc * ns
    b_per_w = B // nw

    src_t3 = init_emb.T.reshape(2, 8, V)
    dst_t3 = output_vecs.T.reshape(2, 8, V)

    src_idx = pairs[:, 0].astype(jnp.int32).reshape(nw, b_per_w)
    dst_idx = pairs[:, 1].astype(jnp.int32).reshape(nw, b_per_w)

    mesh = plsc.VectorSubcoreMesh(core_axis_name="c", subcore_axis_name="s")

    @functools.partial(
        pl.kernel,
        mesh=mesh,
        out_type=jax.ShapeDtypeStruct((B,), jnp.float32),
        compiler_params=pltpu.CompilerParams(needs_layout_passes=False),
        scratch_types=[
            pltpu.VMEM((b_per_w,), jnp.int32),
            pltpu.VMEM((b_per_w,), jnp.int32),
            pltpu.VMEM((2, 8, _RND * 16), jnp.float32),
            pltpu.VMEM((2, 8, _RND * 16), jnp.float32),
            pltpu.VMEM((_RND * 16,), jnp.float32),
            pltpu.VMEM((b_per_w,), jnp.float32),
            pltpu.SemaphoreType.DMA,
        ],
    )
    def run(src_idx_hbm, dst_idx_hbm, src_t3_hbm, dst_t3_hbm, out_hbm,
            sidx_v, didx_v, sblk_v, dblk_v, prod_v, out_v, sem):
        wid = lax.axis_index("s") * nc + lax.axis_index("c")

        pltpu.sync_copy(src_idx_hbm.at[wid], sidx_v)
        pltpu.sync_copy(dst_idx_hbm.at[wid], didx_v)

        iota16 = lax.iota(jnp.int32, 16)
        cb_v = iota16 // 8
        s_v = iota16 % 8

        def round_body(g, _):
            base = g * _RND
            siv = sidx_v[pl.ds(base, _RND)]
            div = didx_v[pl.ds(base, _RND)]
            rs, rd = [], []
            copies = []
            for j in range(_RND):
                r = jnp.clip(jnp.squeeze(lax.slice(siv, (j,), (j + 1,))), 0, V - 1)
                r2 = jnp.clip(jnp.squeeze(lax.slice(div, (j,), (j + 1,))), 0, V - 1)
                rs.append(r)
                rd.append(r2)
                rr = (r // 16) * 16
                rr2 = (r2 // 16) * 16
                for cb in range(2):
                    copies.append(pltpu.async_copy(
                        src_t3_hbm.at[cb, :, pl.ds(rr, 16)],
                        sblk_v.at[cb, :, pl.ds(j * 16, 16)], sem))
                    copies.append(pltpu.async_copy(
                        dst_t3_hbm.at[cb, :, pl.ds(rr2, 16)],
                        dblk_v.at[cb, :, pl.ds(j * 16, 16)], sem))
            for c in copies:
                c.wait()

            for j in range(_RND):
                q = rs[j] % 16
                q2 = rd[j] % 16
                sv = plsc.load_gather(
                    sblk_v, [cb_v, s_v, jnp.full((16,), j * 16, jnp.int32) + q])
                dv = plsc.load_gather(
                    dblk_v, [cb_v, s_v, jnp.full((16,), j * 16, jnp.int32) + q2])
                prod_v[pl.ds(j * 16, 16)] = sv * dv

            acc = jnp.zeros((16,), jnp.float32)
            for c in range(16):
                acc = acc + plsc.load_gather(prod_v, [iota16 * 16 + c])
            prob = 1.0 / (1.0 + jnp.exp(-acc))
            out_v[pl.ds(base, 16)] = prob
            return 0

        lax.fori_loop(0, b_per_w // _RND, round_body, 0)

        pltpu.sync_copy(out_v, out_hbm.at[pl.ds(wid * b_per_w, b_per_w)])

    return run(src_idx, dst_idx, src_t3, dst_t3)


# speedup vs baseline: 7.0356x; 1.1129x over previous
import functools

import jax
import jax.numpy as jnp
from jax import lax
from jax.experimental import pallas as pl
from jax.experimental.pallas import tpu as pltpu
from jax.experimental.pallas import tpu_sc as plsc

_RND = 16


def kernel(pairs, init_emb, output_vecs):
    B = pairs.shape[0]
    V, D = init_emb.shape
    info = plsc.get_sparse_core_info()
    nc, ns = info.num_cores, info.num_subcores
    nw = nc * ns
    b_per_w = B // nw
    n_rounds = b_per_w // _RND

    src_t3 = init_emb.T.reshape(2, 8, V)
    dst_t3 = output_vecs.T.reshape(2, 8, V)

    src_idx = pairs[:, 0].astype(jnp.int32).reshape(nw, b_per_w)
    dst_idx = pairs[:, 1].astype(jnp.int32).reshape(nw, b_per_w)

    mesh = plsc.VectorSubcoreMesh(core_axis_name="c", subcore_axis_name="s")

    @functools.partial(
        pl.kernel,
        mesh=mesh,
        out_type=jax.ShapeDtypeStruct((B,), jnp.float32),
        compiler_params=pltpu.CompilerParams(needs_layout_passes=False),
        scratch_types=[
            pltpu.VMEM((b_per_w,), jnp.int32),
            pltpu.VMEM((b_per_w,), jnp.int32),
            pltpu.VMEM((2, 2, 8, _RND * 16), jnp.float32),
            pltpu.VMEM((2, 2, 8, _RND * 16), jnp.float32),
            pltpu.VMEM((_RND * 16,), jnp.float32),
            pltpu.VMEM((b_per_w,), jnp.float32),
            pltpu.SemaphoreType.DMA,
            pltpu.SemaphoreType.DMA,
        ],
    )
    def run(src_idx_hbm, dst_idx_hbm, src_t3_hbm, dst_t3_hbm, out_hbm,
            sidx_v, didx_v, sblk_v, dblk_v, prod_v, out_v, semA, semB):
        wid = lax.axis_index("s") * nc + lax.axis_index("c")

        pltpu.sync_copy(src_idx_hbm.at[wid], sidx_v)
        pltpu.sync_copy(dst_idx_hbm.at[wid], didx_v)

        iota16 = lax.iota(jnp.int32, 16)
        cb_v = iota16 // 8
        s_v = iota16 % 8

        def issue(rnd, p, sem):
            base = rnd * _RND
            siv = sidx_v[pl.ds(base, _RND)]
            div = didx_v[pl.ds(base, _RND)]
            for j in range(_RND):
                r = jnp.squeeze(lax.slice(siv, (j,), (j + 1,)))
                r2 = jnp.squeeze(lax.slice(div, (j,), (j + 1,)))
                rr = (r // 16) * 16
                rr2 = (r2 // 16) * 16
                pltpu.async_copy(
                    src_t3_hbm.at[:, :, pl.ds(rr, 16)],
                    sblk_v.at[p, :, :, pl.ds(j * 16, 16)], sem)
                pltpu.async_copy(
                    dst_t3_hbm.at[:, :, pl.ds(rr2, 16)],
                    dblk_v.at[p, :, :, pl.ds(j * 16, 16)], sem)

        def drain(p, sem):
            dummy = src_t3_hbm.at[:, :, pl.ds(0, _RND * 16)]
            pltpu.make_async_copy(dummy, sblk_v.at[p], sem).wait()
            pltpu.make_async_copy(dummy, dblk_v.at[p], sem).wait()

        def compute(rnd, p):
            base = rnd * _RND
            siv = sidx_v[pl.ds(base, _RND)]
            div = didx_v[pl.ds(base, _RND)]
            for j in range(_RND):
                q = jnp.squeeze(lax.slice(siv, (j,), (j + 1,))) % 16
                q2 = jnp.squeeze(lax.slice(div, (j,), (j + 1,))) % 16
                col = jnp.full((16,), j * 16, jnp.int32)
                sv = plsc.load_gather(sblk_v.at[p], [cb_v, s_v, col + q])
                dv = plsc.load_gather(dblk_v.at[p], [cb_v, s_v, col + q2])
                prod_v[pl.ds(j * 16, 16)] = sv * dv
            acc = jnp.zeros((16,), jnp.float32)
            for c in range(16):
                acc = acc + plsc.load_gather(prod_v, [iota16 * 16 + c])
            prob = 1.0 / (1.0 + jnp.exp(-acc))
            out_v[pl.ds(base, 16)] = prob

        issue(0, 0, semA)

        def body(i, _):
            g = i * 2
            issue(g + 1, 1, semB)
            drain(0, semA)
            compute(g, 0)

            @pl.when(g + 2 < n_rounds)
            def _():
                issue(g + 2, 0, semA)

            drain(1, semB)
            compute(g + 1, 1)
            return 0

        lax.fori_loop(0, n_rounds // 2, body, 0)

        pltpu.sync_copy(out_v, out_hbm.at[pl.ds(wid * b_per_w, b_per_w)])

    return run(src_idx, dst_idx, src_t3, dst_t3)
